# Initial kernel scaffold; baseline (speedup 1.0000x reference)
import jax
import jax.numpy as jnp
from jax.experimental import pallas as pl


def kernel(hidden_states, gate_w, w1, w3, w2):
    raise NotImplementedError("write your pallas kernel here")


# speedup vs baseline: 2.1185x; 2.1185x over previous
import functools

import jax
import jax.numpy as jnp
from jax import lax
from jax.experimental import pallas as pl
from jax.experimental.pallas import tpu as pltpu

E = 8
TOP_K = 2
D = 1024
FF = 3584
TILE = 256

_INTERPRET = False


def _router_body(x_ref, g_ref, logits_ref, a0_ref, a1_ref, w0_ref, w1_ref):
    x = x_ref[...]
    logits = lax.dot_general(x, g_ref[...], (((1,), (1,)), ((), ())),
                             preferred_element_type=jnp.float32)
    logits_ref[...] = logits
    col = lax.broadcasted_iota(jnp.int32, logits.shape, 1)
    m0 = jnp.max(logits, axis=1, keepdims=True)
    is0 = logits == m0
    a0 = jnp.min(jnp.where(is0, col, E), axis=1, keepdims=True)
    masked = jnp.where(col == a0, -jnp.inf, logits)
    m1 = jnp.max(masked, axis=1, keepdims=True)
    a1 = jnp.min(jnp.where(masked == m1, col, E), axis=1, keepdims=True)
    a0_ref[...] = a0
    a1_ref[...] = a1
    w0_ref[...] = jax.nn.sigmoid(m0 - m1)
    w1_ref[...] = jax.nn.sigmoid(m1 - m0)


def _router(x, gate_w):
    T = x.shape[0]
    TB = 512
    grid = (T // TB,)
    out_shapes = (
        jax.ShapeDtypeStruct((T, E), jnp.float32),
        jax.ShapeDtypeStruct((T, 1), jnp.int32),
        jax.ShapeDtypeStruct((T, 1), jnp.int32),
        jax.ShapeDtypeStruct((T, 1), jnp.float32),
        jax.ShapeDtypeStruct((T, 1), jnp.float32),
    )
    row_spec = pl.BlockSpec((TB, 1), lambda i: (i, 0))
    return pl.pallas_call(
        _router_body,
        grid=grid,
        in_specs=[
            pl.BlockSpec((TB, D), lambda i: (i, 0)),
            pl.BlockSpec((E, D), lambda i: (0, 0)),
        ],
        out_specs=(pl.BlockSpec((TB, E), lambda i: (i, 0)),
                   row_spec, row_spec, row_spec, row_spec),
        out_shape=out_shapes,
        interpret=_INTERPRET,
    )(x, gate_w)


def _dispatch_plan(a0, a1, n_buf, tile):
    T = a0.shape[0]
    ent_e = jnp.stack([a0, a1], axis=1).reshape(-1)
    onehot = (ent_e[:, None] == jnp.arange(E)[None, :]).astype(jnp.int32)
    counts = jnp.sum(onehot, axis=0)
    rank = jnp.take_along_axis(jnp.cumsum(onehot, axis=0) - onehot,
                               ent_e[:, None], axis=1)[:, 0]
    pc = ((counts + tile - 1) // tile) * tile
    pc = pc.at[E - 1].set(n_buf - jnp.sum(pc[: E - 1]))
    ends = jnp.cumsum(pc)
    offsets = ends - pc
    pos = offsets[ent_e] + rank
    tok = jnp.arange(2 * T, dtype=jnp.int32) // 2
    row_token = jnp.zeros((n_buf,), jnp.int32).at[pos].set(tok)
    te = jnp.searchsorted(ends, jnp.arange(0, n_buf, tile), side="right")
    pos2 = pos.reshape(T, 2)
    return pos2[:, 0], pos2[:, 1], row_token, te.astype(jnp.int32)


def _ffn_body(te_ref, x_ref, w1_ref, w3_ref, w2_ref, y_ref):
    x = x_ref[...].astype(jnp.bfloat16)
    a = lax.dot_general(x, w1_ref[0], (((1,), (1,)), ((), ())),
                        preferred_element_type=jnp.float32)
    b = lax.dot_general(x, w3_ref[0], (((1,), (1,)), ((), ())),
                        preferred_element_type=jnp.float32)
    h = (a * jax.nn.sigmoid(a) * b).astype(jnp.bfloat16)
    y_ref[...] = lax.dot_general(h, w2_ref[0], (((1,), (1,)), ((), ())),
                                 preferred_element_type=jnp.float32)


def _ffn(te, xg, w1b, w3b, w2b, n_buf):
    nt = n_buf // TILE
    grid_spec = pltpu.PrefetchScalarGridSpec(
        num_scalar_prefetch=1,
        grid=(nt,),
        in_specs=[
            pl.BlockSpec((TILE, D), lambda i, te: (i, 0)),
            pl.BlockSpec((1, FF, D), lambda i, te: (te[i], 0, 0)),
            pl.BlockSpec((1, FF, D), lambda i, te: (te[i], 0, 0)),
            pl.BlockSpec((1, D, FF), lambda i, te: (te[i], 0, 0)),
        ],
        out_specs=pl.BlockSpec((TILE, D), lambda i, te: (i, 0)),
    )
    return pl.pallas_call(
        _ffn_body,
        grid_spec=grid_spec,
        out_shape=jax.ShapeDtypeStruct((n_buf, D), jnp.float32),
        interpret=_INTERPRET,
    )(te, xg, w1b, w3b, w2b)


def kernel(hidden_states, gate_w, w1, w3, w2):
    B, S, _ = hidden_states.shape
    T = B * S
    n_buf = 2 * T + E * TILE
    x = hidden_states.reshape(T, D)

    logits, a0, a1, w0, w1w = _router(x, gate_w)
    a0, a1 = a0[:, 0], a1[:, 0]
    w0, w1w = w0[:, 0], w1w[:, 0]

    pos0, pos1, row_token, te = _dispatch_plan(a0, a1, n_buf, TILE)

    xg = jnp.take(x, row_token, axis=0)

    w1b = w1.astype(jnp.bfloat16)
    w3b = w3.astype(jnp.bfloat16)
    w2b = w2.astype(jnp.bfloat16)
    y = _ffn(te, xg, w1b, w3b, w2b, n_buf)

    final = (w0[:, None] * jnp.take(y, pos0, axis=0)
             + w1w[:, None] * jnp.take(y, pos1, axis=0))
    return final.reshape(B, S, D), logits


# speedup vs baseline: 2.2456x; 1.0600x over previous
import functools

import jax
import jax.numpy as jnp
from jax import lax
from jax.experimental import pallas as pl
from jax.experimental.pallas import tpu as pltpu

E = 8
TOP_K = 2
D = 1024
FF = 3584
TILE = 256

_INTERPRET = False


def _router_body(x_ref, g_ref, logits_ref, a0_ref, a1_ref, w0_ref, w1_ref):
    x = x_ref[...]
    logits = lax.dot_general(x, g_ref[...], (((1,), (1,)), ((), ())),
                             preferred_element_type=jnp.float32)
    logits_ref[...] = logits
    col = lax.broadcasted_iota(jnp.int32, logits.shape, 1)
    m0 = jnp.max(logits, axis=1, keepdims=True)
    is0 = logits == m0
    a0 = jnp.min(jnp.where(is0, col, E), axis=1, keepdims=True)
    masked = jnp.where(col == a0, -jnp.inf, logits)
    m1 = jnp.max(masked, axis=1, keepdims=True)
    a1 = jnp.min(jnp.where(masked == m1, col, E), axis=1, keepdims=True)
    a0_ref[...] = a0
    a1_ref[...] = a1
    w0_ref[...] = jax.nn.sigmoid(m0 - m1)
    w1_ref[...] = jax.nn.sigmoid(m1 - m0)


def _router(x, gate_w):
    T = x.shape[0]
    TB = 512
    grid = (T // TB,)
    out_shapes = (
        jax.ShapeDtypeStruct((T, E), jnp.float32),
        jax.ShapeDtypeStruct((T, 1), jnp.int32),
        jax.ShapeDtypeStruct((T, 1), jnp.int32),
        jax.ShapeDtypeStruct((T, 1), jnp.float32),
        jax.ShapeDtypeStruct((T, 1), jnp.float32),
    )
    row_spec = pl.BlockSpec((TB, 1), lambda i: (i, 0))
    return pl.pallas_call(
        _router_body,
        grid=grid,
        in_specs=[
            pl.BlockSpec((TB, D), lambda i: (i, 0)),
            pl.BlockSpec((E, D), lambda i: (0, 0)),
        ],
        out_specs=(pl.BlockSpec((TB, E), lambda i: (i, 0)),
                   row_spec, row_spec, row_spec, row_spec),
        out_shape=out_shapes,
        interpret=_INTERPRET,
    )(x, gate_w)


def _dispatch_plan(a0, a1, n_buf, tile):
    T = a0.shape[0]
    ent_e = jnp.stack([a0, a1], axis=1).reshape(-1)
    onehot = (ent_e[:, None] == jnp.arange(E)[None, :]).astype(jnp.int32)
    counts = jnp.sum(onehot, axis=0)
    rank = jnp.take_along_axis(jnp.cumsum(onehot, axis=0) - onehot,
                               ent_e[:, None], axis=1)[:, 0]
    pc = ((counts + tile - 1) // tile) * tile
    pc = pc.at[E - 1].set(n_buf - jnp.sum(pc[: E - 1]))
    ends = jnp.cumsum(pc)
    offsets = ends - pc
    pos = offsets[ent_e] + rank
    tok = jnp.arange(2 * T, dtype=jnp.int32) // 2
    row_token = jnp.zeros((n_buf,), jnp.int32).at[pos].set(tok)
    te = jnp.searchsorted(ends, jnp.arange(0, n_buf, tile), side="right")
    pos2 = pos.reshape(T, 2)
    return pos2[:, 0], pos2[:, 1], row_token, te.astype(jnp.int32)


FFC = 1792


def _h_body(te_ref, x_ref, w1_ref, w3_ref, h_ref):
    x = x_ref[...]
    a = lax.dot_general(x, w1_ref[0], (((1,), (1,)), ((), ())),
                        preferred_element_type=jnp.float32)
    b = lax.dot_general(x, w3_ref[0], (((1,), (1,)), ((), ())),
                        preferred_element_type=jnp.float32)
    h_ref[...] = a * jax.nn.sigmoid(a) * b


def _y_body(te_ref, h_ref, w2_ref, y_ref):
    y_ref[...] = lax.dot_general(h_ref[...], w2_ref[0], (((1,), (1,)), ((), ())),
                                 preferred_element_type=jnp.float32)


def _ffn(te, xg, w1, w3, w2, n_buf):
    nt = n_buf // TILE
    nfc = FF // FFC
    h_spec = pltpu.PrefetchScalarGridSpec(
        num_scalar_prefetch=1,
        grid=(nfc, nt),
        in_specs=[
            pl.BlockSpec((TILE, D), lambda j, i, te: (i, 0)),
            pl.BlockSpec((1, FFC, D), lambda j, i, te: (te[i], j, 0)),
            pl.BlockSpec((1, FFC, D), lambda j, i, te: (te[i], j, 0)),
        ],
        out_specs=pl.BlockSpec((TILE, FFC), lambda j, i, te: (i, j)),
    )
    h = pl.pallas_call(
        _h_body,
        grid_spec=h_spec,
        out_shape=jax.ShapeDtypeStruct((n_buf, FF), jnp.float32),
        interpret=_INTERPRET,
    )(te, xg, w1, w3)
    y_spec = pltpu.PrefetchScalarGridSpec(
        num_scalar_prefetch=1,
        grid=(nt,),
        in_specs=[
            pl.BlockSpec((TILE, FF), lambda i, te: (i, 0)),
            pl.BlockSpec((1, D, FF), lambda i, te: (te[i], 0, 0)),
        ],
        out_specs=pl.BlockSpec((TILE, D), lambda i, te: (i, 0)),
    )
    return pl.pallas_call(
        _y_body,
        grid_spec=y_spec,
        out_shape=jax.ShapeDtypeStruct((n_buf, D), jnp.float32),
        interpret=_INTERPRET,
    )(te, h, w2)


def kernel(hidden_states, gate_w, w1, w3, w2):
    B, S, _ = hidden_states.shape
    T = B * S
    n_buf = 2 * T + E * TILE
    x = hidden_states.reshape(T, D)

    logits, a0, a1, w0, w1w = _router(x, gate_w)
    a0, a1 = a0[:, 0], a1[:, 0]
    w0, w1w = w0[:, 0], w1w[:, 0]

    pos0, pos1, row_token, te = _dispatch_plan(a0, a1, n_buf, TILE)

    xg = jnp.take(x, row_token, axis=0)

    y = _ffn(te, xg, w1, w3, w2, n_buf)

    final = (w0[:, None] * jnp.take(y, pos0, axis=0)
             + w1w[:, None] * jnp.take(y, pos1, axis=0))
    return final.reshape(B, S, D), logits


# speedup vs baseline: 2.2750x; 1.0131x over previous
import functools

import jax
import jax.numpy as jnp
from jax import lax
from jax.experimental import pallas as pl
from jax.experimental.pallas import tpu as pltpu

E = 8
TOP_K = 2
D = 1024
FF = 3584
TILE = 256

_INTERPRET = False


def _router_body(x_ref, g_ref, logits_ref, a0_ref, a1_ref, w0_ref, w1_ref):
    x = x_ref[...]
    logits = lax.dot_general(x, g_ref[...], (((1,), (1,)), ((), ())),
                             preferred_element_type=jnp.float32)
    logits_ref[...] = logits
    col = lax.broadcasted_iota(jnp.int32, logits.shape, 1)
    m0 = jnp.max(logits, axis=1, keepdims=True)
    is0 = logits == m0
    a0 = jnp.min(jnp.where(is0, col, E), axis=1, keepdims=True)
    masked = jnp.where(col == a0, -jnp.inf, logits)
    m1 = jnp.max(masked, axis=1, keepdims=True)
    a1 = jnp.min(jnp.where(masked == m1, col, E), axis=1, keepdims=True)
    a0_ref[...] = a0
    a1_ref[...] = a1
    w0_ref[...] = jax.nn.sigmoid(m0 - m1)
    w1_ref[...] = jax.nn.sigmoid(m1 - m0)


def _router(x, gate_w):
    T = x.shape[0]
    TB = 512
    grid = (T // TB,)
    out_shapes = (
        jax.ShapeDtypeStruct((T, E), jnp.float32),
        jax.ShapeDtypeStruct((T, 1), jnp.int32),
        jax.ShapeDtypeStruct((T, 1), jnp.int32),
        jax.ShapeDtypeStruct((T, 1), jnp.float32),
        jax.ShapeDtypeStruct((T, 1), jnp.float32),
    )
    row_spec = pl.BlockSpec((TB, 1), lambda i: (i, 0))
    return pl.pallas_call(
        _router_body,
        grid=grid,
        in_specs=[
            pl.BlockSpec((TB, D), lambda i: (i, 0)),
            pl.BlockSpec((E, D), lambda i: (0, 0)),
        ],
        out_specs=(pl.BlockSpec((TB, E), lambda i: (i, 0)),
                   row_spec, row_spec, row_spec, row_spec),
        out_shape=out_shapes,
        interpret=_INTERPRET,
    )(x, gate_w)


def _dispatch_plan(a0, a1, n_buf, tile):
    T = a0.shape[0]
    ent_e = jnp.stack([a0, a1], axis=1).reshape(-1)
    onehot = (ent_e[:, None] == jnp.arange(E)[None, :]).astype(jnp.int32)
    counts = jnp.sum(onehot, axis=0)
    rank = jnp.take_along_axis(jnp.cumsum(onehot, axis=0) - onehot,
                               ent_e[:, None], axis=1)[:, 0]
    pc = ((counts + tile - 1) // tile) * tile
    pc = pc.at[E - 1].set(n_buf - jnp.sum(pc[: E - 1]))
    ends = jnp.cumsum(pc)
    offsets = ends - pc
    pos = offsets[ent_e] + rank
    tok = jnp.arange(2 * T, dtype=jnp.int32) // 2
    row_token = jnp.zeros((n_buf,), jnp.int32).at[pos].set(tok)
    te = jnp.searchsorted(ends, jnp.arange(0, n_buf, tile), side="right")
    pos2 = pos.reshape(T, 2)
    return pos2[:, 0], pos2[:, 1], row_token, te.astype(jnp.int32)


FFC = 1792


def _weights_changed(te_ref, i):
    prev = te_ref[jnp.maximum(i - 1, 0)]
    return (i == 0) | (te_ref[i] != prev)


def _h_body(te_ref, x_ref, w1_ref, w3_ref, h_ref, w1s_ref, w3s_ref):
    i = pl.program_id(1)

    @pl.when(_weights_changed(te_ref, i))
    def _():
        w1s_ref[...] = w1_ref[0].astype(jnp.bfloat16)
        w3s_ref[...] = w3_ref[0].astype(jnp.bfloat16)

    x = x_ref[...]
    a = lax.dot_general(x, w1s_ref[...], (((1,), (1,)), ((), ())),
                        preferred_element_type=jnp.float32)
    b = lax.dot_general(x, w3s_ref[...], (((1,), (1,)), ((), ())),
                        preferred_element_type=jnp.float32)
    h_ref[...] = (a * jax.nn.sigmoid(a) * b).astype(jnp.bfloat16)


def _y_body(te_ref, h_ref, w2_ref, y_ref, w2s_ref):
    i = pl.program_id(0)

    @pl.when(_weights_changed(te_ref, i))
    def _():
        w2s_ref[...] = w2_ref[0].astype(jnp.bfloat16)

    y_ref[...] = lax.dot_general(h_ref[...], w2s_ref[...], (((1,), (1,)), ((), ())),
                                 preferred_element_type=jnp.float32)


def _ffn(te, xg, w1, w3, w2, n_buf):
    nt = n_buf // TILE
    nfc = FF // FFC
    h_spec = pltpu.PrefetchScalarGridSpec(
        num_scalar_prefetch=1,
        grid=(nfc, nt),
        in_specs=[
            pl.BlockSpec((TILE, D), lambda j, i, te: (i, 0)),
            pl.BlockSpec((1, FFC, D), lambda j, i, te: (te[i], j, 0)),
            pl.BlockSpec((1, FFC, D), lambda j, i, te: (te[i], j, 0)),
        ],
        out_specs=pl.BlockSpec((TILE, FFC), lambda j, i, te: (i, j)),
        scratch_shapes=[pltpu.VMEM((FFC, D), jnp.bfloat16),
                        pltpu.VMEM((FFC, D), jnp.bfloat16)],
    )
    h = pl.pallas_call(
        _h_body,
        grid_spec=h_spec,
        out_shape=jax.ShapeDtypeStruct((n_buf, FF), jnp.bfloat16),
        interpret=_INTERPRET,
    )(te, xg, w1, w3)
    y_spec = pltpu.PrefetchScalarGridSpec(
        num_scalar_prefetch=1,
        grid=(nt,),
        in_specs=[
            pl.BlockSpec((TILE, FF), lambda i, te: (i, 0)),
            pl.BlockSpec((1, D, FF), lambda i, te: (te[i], 0, 0)),
        ],
        out_specs=pl.BlockSpec((TILE, D), lambda i, te: (i, 0)),
        scratch_shapes=[pltpu.VMEM((D, FF), jnp.bfloat16)],
    )
    return pl.pallas_call(
        _y_body,
        grid_spec=y_spec,
        out_shape=jax.ShapeDtypeStruct((n_buf, D), jnp.float32),
        interpret=_INTERPRET,
    )(te, h, w2)


def kernel(hidden_states, gate_w, w1, w3, w2):
    B, S, _ = hidden_states.shape
    T = B * S
    n_buf = 2 * T + E * TILE
    x = hidden_states.reshape(T, D)

    logits, a0, a1, w0, w1w = _router(x, gate_w)
    a0, a1 = a0[:, 0], a1[:, 0]
    w0, w1w = w0[:, 0], w1w[:, 0]

    pos0, pos1, row_token, te = _dispatch_plan(a0, a1, n_buf, TILE)

    xb = x.astype(jnp.bfloat16)
    xg = jnp.take(xb, row_token, axis=0)

    y = _ffn(te, xg, w1, w3, w2, n_buf)

    final = (w0[:, None] * jnp.take(y, pos0, axis=0)
             + w1w[:, None] * jnp.take(y, pos1, axis=0))
    return final.reshape(B, S, D), logits
